# baseline (speedup 1.0000x reference)
import functools

import jax
import jax.numpy as jnp
from jax import lax
from jax.experimental import pallas as pl
from jax.experimental.pallas import tpu as pltpu
from jax.experimental.pallas import tpu_sc as plsc

_VOCAB = 100000
_DIM = 64
_B = 1024
_M = 200
_T = 4
_BM = _B * _M
_CC = 256

_NSPLIT = 2
_BH = _B // _NSPLIT
_BMH = _BM // _NSPLIT

_NC = 2
_NS = 16
_NW = _NC * _NS
_SLOTS_PER_W = _BMH // _NW
_SCH = 128
_NSCH = _SLOTS_PER_W // _SCH
_TOK = 128
_SUB = _TOK // _T
_NSUB = _SCH // _SUB


def _sc_gather_pool_body(half, story_hbm, ccat_hbm, m01_hbm, m2_hbm,
                         idx_v, stage_v, out01_v, out2_v, gsem, wsem):
    wid = lax.axis_index("s") * _NC + lax.axis_index("c")
    base_slot = wid * _SLOTS_PER_W
    base_chunk = (half * _NW + wid) * _NSCH

    def sch_body(sc, carry):
        pltpu.sync_copy(story_hbm.at[base_chunk + sc], idx_v)
        slot0 = base_slot + sc * _SCH
        gath = [None, None, None]
        wb = [None, None]
        gath[0] = pltpu.async_copy(ccat_hbm.at[idx_v.at[0]],
                                   stage_v.at[0], gsem)
        gath[1] = pltpu.async_copy(ccat_hbm.at[idx_v.at[1]],
                                   stage_v.at[1], gsem)
        for j in range(_NSUB):
            p = j % 3
            if j + 2 < _NSUB:
                gath[(j + 2) % 3] = pltpu.async_copy(
                    ccat_hbm.at[idx_v.at[j + 2]],
                    stage_v.at[(j + 2) % 3], gsem)
            gath[p].wait()
            po = j & 1
            if wb[po] is not None:
                wb[po][0].wait()
                wb[po][1].wait()

            def pool_body(s, c, p=p, po=po):
                r = s * _T
                for q in range(192 // 16):
                    sl = pl.ds(q * 16, 16)
                    acc = (stage_v[p, r, sl] + stage_v[p, r + 1, sl]
                           + stage_v[p, r + 2, sl] + stage_v[p, r + 3, sl])
                    if q < 8:
                        out01_v[po, s, sl] = acc
                    else:
                        out2_v[po, s, pl.ds(q * 16 - 128, 16)] = acc
                return c
            lax.fori_loop(0, _SUB, pool_body, 0)
            dst = slot0 + j * _SUB
            wb[po] = (pltpu.async_copy(out01_v.at[po],
                                       m01_hbm.at[pl.ds(dst, _SUB)], wsem),
                      pltpu.async_copy(out2_v.at[po],
                                       m2_hbm.at[pl.ds(dst, _SUB)], wsem))
        for pair in wb:
            if pair is not None:
                pair[0].wait()
                pair[1].wait()
        return carry

    lax.fori_loop(0, _NSCH, sch_body, 0)


@functools.cache
def _sc_gather_pool(half):
    return pl.kernel(
        functools.partial(_sc_gather_pool_body, half),
        mesh=plsc.VectorSubcoreMesh(core_axis_name="c", subcore_axis_name="s",
                                    num_cores=_NC, num_subcores=_NS),
        out_type=[jax.ShapeDtypeStruct((_BMH, 128), jnp.float32),
                  jax.ShapeDtypeStruct((_BMH, _DIM), jnp.float32)],
        scratch_types=[
            pltpu.VMEM((_NSUB, 128), jnp.int32),
            pltpu.VMEM((3, _TOK, _CC), jnp.float32),
            pltpu.VMEM((2, _SUB, 128), jnp.float32),
            pltpu.VMEM((2, _SUB, _DIM), jnp.float32),
            pltpu.SemaphoreType.DMA,
            pltpu.SemaphoreType.DMA,
        ],
    )


def _attn_body(q_ref, m01_ref, m2_ref, pptr_ref, u1_ref):
    bb = q_ref.shape[0]
    m01 = m01_ref[...].reshape(bb, _M, 128)
    m2 = m2_ref[...].reshape(bb, _M, _DIM)
    u0 = q_ref[...]
    z = jnp.zeros_like(u0)
    u0p = jnp.concatenate([u0, z], axis=1)
    lg0 = jnp.sum(m01 * u0p[:, None, :], axis=2)
    p0 = jax.nn.softmax(lg0, axis=1)
    oo = jnp.sum(m01 * p0[:, :, None], axis=1)
    u1 = u0 + oo[:, _DIM:]
    u1_ref[...] = u1
    u1p = jnp.concatenate([z, u1], axis=1)
    lg1 = jnp.sum(m01 * u1p[:, None, :], axis=2)
    p1 = jax.nn.softmax(lg1, axis=1)
    o1 = jnp.sum(m2 * p1[:, :, None], axis=1)
    u2 = u1 + o1
    pptr_ref[...] = jnp.sum(m2 * u2[:, None, :], axis=2)


def _tc_attention(enc_query, m01, m2):
    bb = 64
    grid = (_BH // bb,)
    return pl.pallas_call(
        _attn_body,
        grid=grid,
        in_specs=[pl.BlockSpec((bb, _DIM), lambda i: (i, 0)),
                  pl.BlockSpec((bb * _M, 128), lambda i: (i, 0)),
                  pl.BlockSpec((bb * _M, _DIM), lambda i: (i, 0))],
        out_specs=[pl.BlockSpec((bb, _M), lambda i: (i, 0)),
                   pl.BlockSpec((bb, _DIM), lambda i: (i, 0))],
        out_shape=[jax.ShapeDtypeStruct((_BH, _M), jnp.float32),
                   jax.ShapeDtypeStruct((_BH, _DIM), jnp.float32)],
    )(enc_query, m01, m2)


def _ccat_body(c0_ref, c1_ref, c2_ref, out_ref):
    c0 = c0_ref[0]
    out_ref[...] = jnp.concatenate(
        [c0, c1_ref[0], c2_ref[0], jnp.zeros_like(c0)], axis=1)


def _build_ccat(C):
    vb = 5000
    grid = (_VOCAB // vb,)
    return pl.pallas_call(
        _ccat_body,
        grid=grid,
        in_specs=[pl.BlockSpec((1, vb, _DIM), lambda i, h=h: (h, i, 0))
                  for h in range(3)],
        out_specs=pl.BlockSpec((vb, _CC), lambda i: (i, 0)),
        out_shape=jax.ShapeDtypeStruct((_VOCAB, _CC), jnp.float32),
    )(C, C, C)


def kernel(story, enc_query, C):
    story3d = story.astype(jnp.int32).reshape(
        _BM // _SCH, _NSUB, 128)
    ccat = _build_ccat(C)
    pptrs, u1s = [], []
    for h in range(_NSPLIT):
        m01, m2 = _sc_gather_pool(h)(story3d, ccat)
        pp, u1 = _tc_attention(enc_query[h * _BH:(h + 1) * _BH], m01, m2)
        pptrs.append(pp)
        u1s.append(u1)
    return (jnp.concatenate(pptrs, axis=0), jnp.concatenate(u1s, axis=0))


# speedup vs baseline: 1.0437x; 1.0437x over previous
import functools

import jax
import jax.numpy as jnp
from jax import lax
from jax.experimental import pallas as pl
from jax.experimental.pallas import tpu as pltpu
from jax.experimental.pallas import tpu_sc as plsc

_VOCAB = 100000
_DIM = 64
_B = 1024
_M = 200
_T = 4
_BM = _B * _M
_CC = 256

_NSPLIT = 2
_BH = _B // _NSPLIT
_BMH = _BM // _NSPLIT

_NC = 2
_NS = 16
_NW = _NC * _NS
_SLOTS_PER_W = _BMH // _NW
_SCH = 128
_NSCH = _SLOTS_PER_W // _SCH
_TOK = 128
_SUB = _TOK // _T
_NSUB = _SCH // _SUB


def _sc_gather_pool_body(half, story_hbm, ccat_hbm, m01_hbm, m2_hbm,
                         idx_v, stage_v, out01_v, out2_v, gsem, wsem):
    wid = lax.axis_index("s") * _NC + lax.axis_index("c")
    base_slot = wid * _SLOTS_PER_W
    base_chunk = (half * _NW + wid) * _NSCH

    def sch_body(sc, carry):
        pltpu.sync_copy(story_hbm.at[base_chunk + sc], idx_v)
        slot0 = base_slot + sc * _SCH
        gath = [None, None, None]
        wb = [None, None]
        gath[0] = pltpu.async_copy(ccat_hbm.at[idx_v.at[0]],
                                   stage_v.at[0], gsem)
        gath[1] = pltpu.async_copy(ccat_hbm.at[idx_v.at[1]],
                                   stage_v.at[1], gsem)
        for j in range(_NSUB):
            p = j % 3
            if j + 2 < _NSUB:
                gath[(j + 2) % 3] = pltpu.async_copy(
                    ccat_hbm.at[idx_v.at[j + 2]],
                    stage_v.at[(j + 2) % 3], gsem)
            gath[p].wait()
            po = j & 1
            if wb[po] is not None:
                wb[po][0].wait()
                wb[po][1].wait()

            def pool_body(s, c, p=p, po=po):
                r = s * _T
                for q in range(192 // 16):
                    sl = pl.ds(q * 16, 16)
                    acc = (stage_v[p, r, sl] + stage_v[p, r + 1, sl]
                           + stage_v[p, r + 2, sl] + stage_v[p, r + 3, sl])
                    if q < 8:
                        out01_v[po, s, sl] = acc
                    else:
                        out2_v[po, s, pl.ds(q * 16 - 128, 16)] = acc
                return c
            lax.fori_loop(0, _SUB, pool_body, 0)
            dst = slot0 + j * _SUB
            wb[po] = (pltpu.async_copy(out01_v.at[po],
                                       m01_hbm.at[pl.ds(dst, _SUB)], wsem),
                      pltpu.async_copy(out2_v.at[po],
                                       m2_hbm.at[pl.ds(dst, _SUB)], wsem))
        for pair in wb:
            if pair is not None:
                pair[0].wait()
                pair[1].wait()
        return carry

    lax.fori_loop(0, _NSCH, sch_body, 0)


@functools.cache
def _sc_gather_pool(half):
    return pl.kernel(
        functools.partial(_sc_gather_pool_body, half),
        mesh=plsc.VectorSubcoreMesh(core_axis_name="c", subcore_axis_name="s",
                                    num_cores=_NC, num_subcores=_NS),
        out_type=[jax.ShapeDtypeStruct((_BMH, 128), jnp.float32),
                  jax.ShapeDtypeStruct((_BMH, _DIM), jnp.float32)],
        scratch_types=[
            pltpu.VMEM((_NSUB, 128), jnp.int32),
            pltpu.VMEM((3, _TOK, _CC), jnp.float32),
            pltpu.VMEM((2, _SUB, 128), jnp.float32),
            pltpu.VMEM((2, _SUB, _DIM), jnp.float32),
            pltpu.SemaphoreType.DMA,
            pltpu.SemaphoreType.DMA,
        ],
    )


def _attn_body(q_ref, m01_ref, m2_ref, pptr_ref, u1_ref):
    bb = q_ref.shape[0]
    m01 = m01_ref[...].reshape(bb, _M, 128)
    m2 = m2_ref[...].reshape(bb, _M, _DIM)
    u0 = q_ref[...]
    z = jnp.zeros_like(u0)
    u0p = jnp.concatenate([u0, z], axis=1)
    lg0 = jnp.sum(m01 * u0p[:, None, :], axis=2)
    p0 = jax.nn.softmax(lg0, axis=1)
    oo = jnp.sum(m01 * p0[:, :, None], axis=1)
    u1 = u0 + oo[:, _DIM:]
    u1_ref[...] = u1
    u1p = jnp.concatenate([z, u1], axis=1)
    lg1 = jnp.sum(m01 * u1p[:, None, :], axis=2)
    p1 = jax.nn.softmax(lg1, axis=1)
    o1 = jnp.sum(m2 * p1[:, :, None], axis=1)
    u2 = u1 + o1
    pptr_ref[...] = jnp.sum(m2 * u2[:, None, :], axis=2)


def _tc_attention(enc_query, m01, m2):
    bb = 64
    grid = (_BH // bb,)
    return pl.pallas_call(
        _attn_body,
        grid=grid,
        in_specs=[pl.BlockSpec((bb, _DIM), lambda i: (i, 0)),
                  pl.BlockSpec((bb * _M, 128), lambda i: (i, 0)),
                  pl.BlockSpec((bb * _M, _DIM), lambda i: (i, 0))],
        out_specs=[pl.BlockSpec((bb, _M), lambda i: (i, 0)),
                   pl.BlockSpec((bb, _DIM), lambda i: (i, 0))],
        out_shape=[jax.ShapeDtypeStruct((_BH, _M), jnp.float32),
                   jax.ShapeDtypeStruct((_BH, _DIM), jnp.float32)],
    )(enc_query, m01, m2)


def _ccat_body(c0_ref, c1_ref, c2_ref, out_ref):
    c0 = c0_ref[0]
    out_ref[...] = jnp.concatenate(
        [c0, c1_ref[0], c2_ref[0], jnp.zeros_like(c0)], axis=1)


def _build_ccat(C):
    vb = 5000
    grid = (_VOCAB // vb,)
    return pl.pallas_call(
        _ccat_body,
        grid=grid,
        in_specs=[pl.BlockSpec((1, vb, _DIM), lambda i, h=h: (h, i, 0))
                  for h in range(3)],
        out_specs=pl.BlockSpec((vb, _CC), lambda i: (i, 0)),
        out_shape=jax.ShapeDtypeStruct((_VOCAB, _CC), jnp.float32),
    )(C, C, C)


def kernel(story, enc_query, C):
    story3d = story.astype(jnp.int32).reshape(
        _NSPLIT, _BMH // _SCH, _NSUB, 128)
    ccat = _build_ccat(C)
    pptrs, u1s = [], []
    for h in range(_NSPLIT):
        m01, m2 = _sc_gather_pool(0)(story3d[h], ccat)
        pp, u1 = _tc_attention(enc_query[h * _BH:(h + 1) * _BH], m01, m2)
        pptrs.append(pp)
        u1s.append(u1)
    return (jnp.concatenate(pptrs, axis=0), jnp.concatenate(u1s, axis=0))


# speedup vs baseline: 1.1521x; 1.1039x over previous
import functools

import jax
import jax.numpy as jnp
from jax import lax
from jax.experimental import pallas as pl
from jax.experimental.pallas import tpu as pltpu
from jax.experimental.pallas import tpu_sc as plsc

_VOCAB = 100000
_DIM = 64
_B = 1024
_M = 200
_T = 4
_BM = _B * _M
_CC = 128

_NSPLIT = 2
_BH = _B // _NSPLIT
_BMH = _BM // _NSPLIT

_NC = 2
_NS = 16
_NW = _NC * _NS
_SLOTS_PER_W = _BMH // _NW
_SCH = 128
_NSCH = _SLOTS_PER_W // _SCH
_TOK = 128
_SUB = _TOK // _T
_NSUB = _SCH // _SUB


def _sc_gather_pool_body(story_hbm, ccat_hbm, m01_hbm,
                         idx_v, stage_v, out01_v, gsem, wsem):
    wid = lax.axis_index("s") * _NC + lax.axis_index("c")
    base_slot = wid * _SLOTS_PER_W
    base_chunk = wid * _NSCH

    def sch_body(sc, carry):
        pltpu.sync_copy(story_hbm.at[base_chunk + sc], idx_v)
        slot0 = base_slot + sc * _SCH
        gath = [None, None, None]
        wb = [None, None]
        gath[0] = pltpu.async_copy(ccat_hbm.at[idx_v.at[0]],
                                   stage_v.at[0], gsem)
        gath[1] = pltpu.async_copy(ccat_hbm.at[idx_v.at[1]],
                                   stage_v.at[1], gsem)
        for j in range(_NSUB):
            p = j % 3
            if j + 2 < _NSUB:
                gath[(j + 2) % 3] = pltpu.async_copy(
                    ccat_hbm.at[idx_v.at[j + 2]],
                    stage_v.at[(j + 2) % 3], gsem)
            gath[p].wait()
            po = j & 1
            if wb[po] is not None:
                wb[po].wait()

            def pool_body(s, c, p=p, po=po):
                r = s * _T
                for q in range(4):
                    sl = pl.ds(q * 16, 16)
                    out01_v[po, s, sl] = (
                        stage_v[p, r, sl] + stage_v[p, r + 1, sl]
                        + stage_v[p, r + 2, sl] + stage_v[p, r + 3, sl])
                hi = jnp.uint32(0xFFFF0000)
                for q in range(4, 8):
                    sl = pl.ds(q * 16, 16)
                    m1a = None
                    m2a = None
                    for t in range(_T):
                        x = plsc.bitcast(stage_v[p, r + t, sl], jnp.uint32)
                        a = plsc.bitcast(x << 16, jnp.float32)
                        b = plsc.bitcast(x & hi, jnp.float32)
                        m1a = a if m1a is None else m1a + a
                        m2a = b if m2a is None else m2a + b
                    rnd = jnp.uint32(0x8000)
                    out01_v[po, s, sl] = plsc.bitcast(
                        ((plsc.bitcast(m1a, jnp.uint32) + rnd) >> 16)
                        | ((plsc.bitcast(m2a, jnp.uint32) + rnd) & hi),
                        jnp.float32)
                return c
            lax.fori_loop(0, _SUB, pool_body, 0)
            dst = slot0 + j * _SUB
            wb[po] = pltpu.async_copy(out01_v.at[po],
                                      m01_hbm.at[pl.ds(dst, _SUB)], wsem)
        for d in wb:
            if d is not None:
                d.wait()
        return carry

    lax.fori_loop(0, _NSCH, sch_body, 0)


@functools.cache
def _sc_gather_pool():
    return pl.kernel(
        _sc_gather_pool_body,
        mesh=plsc.VectorSubcoreMesh(core_axis_name="c", subcore_axis_name="s",
                                    num_cores=_NC, num_subcores=_NS),
        out_type=jax.ShapeDtypeStruct((_BMH, _CC), jnp.float32),
        scratch_types=[
            pltpu.VMEM((_NSUB, 128), jnp.int32),
            pltpu.VMEM((3, _TOK, _CC), jnp.float32),
            pltpu.VMEM((2, _SUB, _CC), jnp.float32),
            pltpu.SemaphoreType.DMA,
            pltpu.SemaphoreType.DMA,
        ],
        compiler_params=pltpu.CompilerParams(needs_layout_passes=False),
    )


def _attn_body(q_ref, mp_ref, pptr_ref, u1_ref):
    bb = q_ref.shape[0]
    mp = mp_ref[...].reshape(bb, _M, _CC)
    m0 = mp[:, :, :_DIM]
    u = jax.lax.bitcast_convert_type(mp[:, :, _DIM:], jnp.uint32)
    m1 = jax.lax.bitcast_convert_type(u << 16, jnp.float32)
    m2 = jax.lax.bitcast_convert_type(u & jnp.uint32(0xFFFF0000),
                                      jnp.float32)
    u0 = q_ref[...]
    lg0 = jnp.sum(m0 * u0[:, None, :], axis=2)
    p0 = jax.nn.softmax(lg0, axis=1)
    o0 = jnp.sum(m1 * p0[:, :, None], axis=1)
    u1 = u0 + o0
    u1_ref[...] = u1
    lg1 = jnp.sum(m1 * u1[:, None, :], axis=2)
    p1 = jax.nn.softmax(lg1, axis=1)
    o1 = jnp.sum(m2 * p1[:, :, None], axis=1)
    u2 = u1 + o1
    pptr_ref[...] = jnp.sum(m2 * u2[:, None, :], axis=2)


def _tc_attention(enc_query, mp):
    bb = 64
    grid = (_BH // bb,)
    return pl.pallas_call(
        _attn_body,
        grid=grid,
        in_specs=[pl.BlockSpec((bb, _DIM), lambda i: (i, 0)),
                  pl.BlockSpec((bb * _M, _CC), lambda i: (i, 0))],
        out_specs=[pl.BlockSpec((bb, _M), lambda i: (i, 0)),
                   pl.BlockSpec((bb, _DIM), lambda i: (i, 0))],
        out_shape=[jax.ShapeDtypeStruct((_BH, _M), jnp.float32),
                   jax.ShapeDtypeStruct((_BH, _DIM), jnp.float32)],
    )(enc_query, mp)


def _ccat_body(c0_ref, c1_ref, c2_ref, out_ref):
    rnd = jnp.uint32(0x8000)
    b1 = (jax.lax.bitcast_convert_type(c1_ref[0], jnp.uint32) + rnd) >> 16
    b2 = ((jax.lax.bitcast_convert_type(c2_ref[0], jnp.uint32) + rnd)
          & jnp.uint32(0xFFFF0000))
    packed = jax.lax.bitcast_convert_type(b1 | b2, jnp.float32)
    out_ref[...] = jnp.concatenate([c0_ref[0], packed], axis=1)


def _build_ccat(C):
    vb = 5000
    grid = (_VOCAB // vb,)
    return pl.pallas_call(
        _ccat_body,
        grid=grid,
        in_specs=[pl.BlockSpec((1, vb, _DIM), lambda i, h=h: (h, i, 0))
                  for h in range(3)],
        out_specs=pl.BlockSpec((vb, _CC), lambda i: (i, 0)),
        out_shape=jax.ShapeDtypeStruct((_VOCAB, _CC), jnp.float32),
    )(C, C, C)


def kernel(story, enc_query, C):
    story3d = story.astype(jnp.int32).reshape(
        _NSPLIT, _BMH // _SCH, _NSUB, 128)
    ccat = _build_ccat(C)
    pptrs, u1s = [], []
    for h in range(_NSPLIT):
        mp = _sc_gather_pool()(story3d[h], ccat)
        pp, u1 = _tc_attention(enc_query[h * _BH:(h + 1) * _BH], mp)
        pptrs.append(pp)
        u1s.append(u1)
    return (jnp.concatenate(pptrs, axis=0), jnp.concatenate(u1s, axis=0))
